# Initial kernel scaffold; baseline (speedup 1.0000x reference)
import functools

import jax
import jax.numpy as jnp
from jax import lax
from jax.experimental import pallas as pl
from jax.experimental.pallas import tpu as pltpu
from jax.experimental.pallas import tpu_sc as plsc

N_NODES = 10000
N_EDGES = 160000
D_IN = 256
D_OUT = 128

NC = 2
NS = 16
NW = NC * NS
EPW = N_EDGES // NW
CHUNK = 128
NFULL = EPW // CHUNK
TAIL = EPW - NFULL * CHUNK
RPS = N_NODES // NS
RCH = 125
DEG_W = 16

_mesh = plsc.VectorSubcoreMesh(core_axis_name="c", subcore_axis_name="s")


def _fill(ref, val):
    rows, cols = ref.shape
    z = jnp.full((16,), val, ref.dtype)

    def body(r, carry):
        for j in range(cols // 16):
            ref[r, pl.ds(j * 16, 16)] = z
        return carry

    lax.fori_loop(0, rows, body, 0)


def _deg_body(dst_hbm, out_hbm, idx_v, idx_t, ones_v, stage, acc):
    c = lax.axis_index("c")
    s = lax.axis_index("s")
    w = c * NS + s

    _fill(ones_v, 1.0)
    _fill(stage, 0.0)
    for r in range(RPS // RCH):
        pltpu.sync_copy(stage, acc.at[pl.ds(s * RPS + r * RCH, RCH)])
    plsc.subcore_barrier()

    base0 = w * EPW

    def chunk(i, carry):
        pltpu.sync_copy(dst_hbm.at[pl.ds(base0 + i * CHUNK, CHUNK)], idx_v)
        pltpu.sync_copy(ones_v, acc.at[idx_v], add=True)
        return carry

    lax.fori_loop(0, NFULL, chunk, 0)
    pltpu.sync_copy(dst_hbm.at[pl.ds(base0 + NFULL * CHUNK, TAIL)], idx_t)
    pltpu.sync_copy(ones_v.at[pl.ds(0, TAIL)], acc.at[idx_t], add=True)
    plsc.subcore_barrier()

    for r in range(RPS // RCH):
        off = s * RPS + r * RCH
        pltpu.sync_copy(acc.at[pl.ds(off, RCH)], stage)
        pltpu.sync_copy(stage, out_hbm.at[c, pl.ds(off, RCH)])


_deg = pl.kernel(
    _deg_body,
    out_type=jax.ShapeDtypeStruct((NC, N_NODES, DEG_W), jnp.float32),
    mesh=_mesh,
    scratch_types=[
        pltpu.VMEM((CHUNK,), jnp.int32),
        pltpu.VMEM((TAIL,), jnp.int32),
        pltpu.VMEM((CHUNK, DEG_W), jnp.float32),
        pltpu.VMEM((RCH, DEG_W), jnp.float32),
        pltpu.VMEM_SHARED((N_NODES, DEG_W), jnp.float32),
    ],
)


MBLK = 256
GRID_M = (N_NODES + MBLK - 1) // MBLK


def _mm_body(x_ref, w_ref, dp_ref, g_ref):
    h = jnp.dot(x_ref[...], w_ref[...], preferred_element_type=jnp.float32)
    deg = dp_ref[0, :, 0] + dp_ref[1, :, 0] + 1.0
    dinv = lax.rsqrt(deg)
    g_ref[...] = h * dinv[:, None]


def _mm(x, W, dp):
    return pl.pallas_call(
        _mm_body,
        grid=(GRID_M,),
        in_specs=[
            pl.BlockSpec((MBLK, D_IN), lambda i: (i, 0)),
            pl.BlockSpec((D_IN, D_OUT), lambda i: (0, 0)),
            pl.BlockSpec((NC, MBLK, DEG_W), lambda i: (0, i, 0)),
        ],
        out_specs=pl.BlockSpec((MBLK, D_OUT), lambda i: (i, 0)),
        out_shape=jax.ShapeDtypeStruct((N_NODES, D_OUT), jnp.float32),
    )(x, W, dp)


def _agg_body(g_hbm, src_hbm, dst_hbm, out_hbm,
              sidx, didx, sidx_t, didx_t, rows, rows_t, stage, acc, sem):
    c = lax.axis_index("c")
    s = lax.axis_index("s")
    w = c * NS + s

    _fill(stage, 0.0)
    for r in range(RPS // RCH):
        pltpu.sync_copy(stage, acc.at[pl.ds(s * RPS + r * RCH, RCH)])
    plsc.subcore_barrier()

    base0 = w * EPW

    def chunk(i, carry):
        base = base0 + i * CHUNK
        pltpu.sync_copy(src_hbm.at[pl.ds(base, CHUNK)], sidx)
        pltpu.sync_copy(dst_hbm.at[pl.ds(base, CHUNK)], didx)
        pltpu.async_copy(g_hbm.at[sidx], rows, sem).wait()
        pltpu.sync_copy(rows, acc.at[didx], add=True)
        return carry

    lax.fori_loop(0, NFULL, chunk, 0)

    tbase = base0 + NFULL * CHUNK
    pltpu.sync_copy(src_hbm.at[pl.ds(tbase, TAIL)], sidx_t)
    pltpu.sync_copy(dst_hbm.at[pl.ds(tbase, TAIL)], didx_t)
    pltpu.async_copy(g_hbm.at[sidx_t], rows_t, sem).wait()
    pltpu.sync_copy(rows_t, acc.at[didx_t], add=True)
    plsc.subcore_barrier()

    for r in range(RPS // RCH):
        off = s * RPS + r * RCH
        pltpu.sync_copy(acc.at[pl.ds(off, RCH)], stage)
        pltpu.sync_copy(stage, out_hbm.at[c, pl.ds(off, RCH)])


_agg = pl.kernel(
    _agg_body,
    out_type=jax.ShapeDtypeStruct((NC, N_NODES, D_OUT), jnp.float32),
    mesh=_mesh,
    scratch_types=[
        pltpu.VMEM((CHUNK,), jnp.int32),
        pltpu.VMEM((CHUNK,), jnp.int32),
        pltpu.VMEM((TAIL,), jnp.int32),
        pltpu.VMEM((TAIL,), jnp.int32),
        pltpu.VMEM((CHUNK, D_OUT), jnp.float32),
        pltpu.VMEM((TAIL, D_OUT), jnp.float32),
        pltpu.VMEM((RCH, D_OUT), jnp.float32),
        pltpu.VMEM_SHARED((N_NODES, D_OUT), jnp.float32),
        pltpu.SemaphoreType.DMA,
    ],
)


def _fin_body(p_ref, g_ref, dp_ref, b_ref, o_ref):
    deg = dp_ref[0, :, 0] + dp_ref[1, :, 0] + 1.0
    dinv = lax.rsqrt(deg)
    o_ref[...] = dinv[:, None] * (p_ref[0] + p_ref[1] + g_ref[...]) + b_ref[...]


def _fin(P, g, dp, b2):
    return pl.pallas_call(
        _fin_body,
        grid=(GRID_M,),
        in_specs=[
            pl.BlockSpec((NC, MBLK, D_OUT), lambda i: (0, i, 0)),
            pl.BlockSpec((MBLK, D_OUT), lambda i: (i, 0)),
            pl.BlockSpec((NC, MBLK, DEG_W), lambda i: (0, i, 0)),
            pl.BlockSpec((1, D_OUT), lambda i: (0, 0)),
        ],
        out_specs=pl.BlockSpec((MBLK, D_OUT), lambda i: (i, 0)),
        out_shape=jax.ShapeDtypeStruct((N_NODES, D_OUT), jnp.float32),
    )(P, g, dp, b2)


def kernel(x, edge_index, W, b):
    ei = edge_index.astype(jnp.int32)
    src = ei[0]
    dst = ei[1]
    dp = _deg(dst)
    g = _mm(x, W, dp)
    P = _agg(g, src, dst)
    return _fin(P, g, dp, b.reshape(1, D_OUT))


# speedup vs baseline: 14.6932x; 14.6932x over previous
import functools

import jax
import jax.numpy as jnp
from jax import lax
from jax.experimental import pallas as pl
from jax.experimental.pallas import tpu as pltpu
from jax.experimental.pallas import tpu_sc as plsc

N_NODES = 10000
N_EDGES = 160000
D_IN = 256
D_OUT = 128

NC = 2
NS = 16
NW = NC * NS
EPW = N_EDGES // NW
CHUNK = 128
NFULL = EPW // CHUNK
TAIL = EPW - NFULL * CHUNK
NPAD = 10240
RPS = NPAD // NS
RCH = 128
DEG_W = 16

_mesh = plsc.VectorSubcoreMesh(core_axis_name="c", subcore_axis_name="s")


def _fill(ref, val):
    rows, cols = ref.shape
    z = jnp.full((16,), val, ref.dtype)

    def body(r, carry):
        for j in range(cols // 16):
            ref[r, pl.ds(j * 16, 16)] = z
        return carry

    lax.fori_loop(0, rows, body, 0)


def _deg_body(dst_hbm, out_hbm, idx_v, idx_t, ones_v, stage, acc):
    c = lax.axis_index("c")
    s = lax.axis_index("s")
    w = c * NS + s

    _fill(ones_v, 1.0)
    _fill(stage, 0.0)
    for r in range(RPS // RCH):
        pltpu.sync_copy(stage, acc.at[pl.ds(s * RPS + r * RCH, RCH)])
    plsc.subcore_barrier()

    base0 = w * EPW

    def chunk(i, carry):
        pltpu.sync_copy(dst_hbm.at[pl.ds(base0 + i * CHUNK, CHUNK)], idx_v)
        pltpu.sync_copy(ones_v, acc.at[idx_v], add=True)
        return carry

    lax.fori_loop(0, NFULL, chunk, 0)
    pltpu.sync_copy(dst_hbm.at[pl.ds(base0 + NFULL * CHUNK, TAIL)], idx_t)
    pltpu.sync_copy(ones_v.at[pl.ds(0, TAIL)], acc.at[idx_t], add=True)
    plsc.subcore_barrier()

    for r in range(RPS // RCH):
        off = s * RPS + r * RCH
        pltpu.sync_copy(acc.at[pl.ds(off, RCH)], stage)
        pltpu.sync_copy(stage, out_hbm.at[c, pl.ds(off, RCH)])


_deg = pl.kernel(
    _deg_body,
    out_type=jax.ShapeDtypeStruct((NC, NPAD, DEG_W), jnp.float32),
    mesh=_mesh,
    scratch_types=[
        pltpu.VMEM((CHUNK,), jnp.int32),
        pltpu.VMEM((TAIL,), jnp.int32),
        pltpu.VMEM((CHUNK, DEG_W), jnp.float32),
        pltpu.VMEM((RCH, DEG_W), jnp.float32),
        pltpu.VMEM_SHARED((NPAD, DEG_W), jnp.float32),
    ],
)


MBLK = 256
GRID_M = (N_NODES + MBLK - 1) // MBLK


def _mm_body(x_ref, w_ref, dp_ref, g_ref):
    h = jnp.dot(x_ref[...], w_ref[...], preferred_element_type=jnp.float32)
    deg = dp_ref[0, :, 0] + dp_ref[1, :, 0] + 1.0
    dinv = lax.rsqrt(deg)
    g_ref[...] = h * dinv[:, None]


def _mm(x, W, dp):
    return pl.pallas_call(
        _mm_body,
        grid=(GRID_M,),
        in_specs=[
            pl.BlockSpec((MBLK, D_IN), lambda i: (i, 0)),
            pl.BlockSpec((D_IN, D_OUT), lambda i: (0, 0)),
            pl.BlockSpec((NC, MBLK, DEG_W), lambda i: (0, i, 0)),
        ],
        out_specs=pl.BlockSpec((MBLK, D_OUT), lambda i: (i, 0)),
        out_shape=jax.ShapeDtypeStruct((N_NODES, D_OUT), jnp.float32),
    )(x, W, dp)


def _agg_body(g_hbm, src_hbm, dst_hbm, out_hbm,
              sidx, didx, sidx_t, didx_t, rows, rows_t, stage, acc, sem):
    c = lax.axis_index("c")
    s = lax.axis_index("s")
    w = c * NS + s

    _fill(stage, 0.0)
    for r in range(RPS // RCH):
        pltpu.sync_copy(stage, acc.at[pl.ds(s * RPS + r * RCH, RCH)])
    plsc.subcore_barrier()

    base0 = w * EPW

    def chunk(i, carry):
        base = base0 + i * CHUNK
        pltpu.sync_copy(src_hbm.at[pl.ds(base, CHUNK)], sidx)
        pltpu.sync_copy(dst_hbm.at[pl.ds(base, CHUNK)], didx)
        pltpu.async_copy(g_hbm.at[sidx], rows, sem).wait()
        pltpu.sync_copy(rows, acc.at[didx], add=True)
        return carry

    lax.fori_loop(0, NFULL, chunk, 0)

    tbase = base0 + NFULL * CHUNK
    pltpu.sync_copy(src_hbm.at[pl.ds(tbase, TAIL)], sidx_t)
    pltpu.sync_copy(dst_hbm.at[pl.ds(tbase, TAIL)], didx_t)
    pltpu.async_copy(g_hbm.at[sidx_t], rows_t, sem).wait()
    pltpu.sync_copy(rows_t, acc.at[didx_t], add=True)
    plsc.subcore_barrier()

    for r in range(RPS // RCH):
        off = s * RPS + r * RCH
        pltpu.sync_copy(acc.at[pl.ds(off, RCH)], stage)
        pltpu.sync_copy(stage, out_hbm.at[c, pl.ds(off, RCH)])


_agg = pl.kernel(
    _agg_body,
    out_type=jax.ShapeDtypeStruct((NC, NPAD, D_OUT), jnp.float32),
    mesh=_mesh,
    scratch_types=[
        pltpu.VMEM((CHUNK,), jnp.int32),
        pltpu.VMEM((CHUNK,), jnp.int32),
        pltpu.VMEM((TAIL,), jnp.int32),
        pltpu.VMEM((TAIL,), jnp.int32),
        pltpu.VMEM((CHUNK, D_OUT), jnp.float32),
        pltpu.VMEM((TAIL, D_OUT), jnp.float32),
        pltpu.VMEM((RCH, D_OUT), jnp.float32),
        pltpu.VMEM_SHARED((NPAD, D_OUT), jnp.float32),
        pltpu.SemaphoreType.DMA,
    ],
)


def _fin_body(p_ref, g_ref, dp_ref, b_ref, o_ref):
    deg = dp_ref[0, :, 0] + dp_ref[1, :, 0] + 1.0
    dinv = lax.rsqrt(deg)
    o_ref[...] = dinv[:, None] * (p_ref[0] + p_ref[1] + g_ref[...]) + b_ref[...]


def _fin(P, g, dp, b2):
    return pl.pallas_call(
        _fin_body,
        grid=(GRID_M,),
        in_specs=[
            pl.BlockSpec((NC, MBLK, D_OUT), lambda i: (0, i, 0)),
            pl.BlockSpec((MBLK, D_OUT), lambda i: (i, 0)),
            pl.BlockSpec((NC, MBLK, DEG_W), lambda i: (0, i, 0)),
            pl.BlockSpec((1, D_OUT), lambda i: (0, 0)),
        ],
        out_specs=pl.BlockSpec((MBLK, D_OUT), lambda i: (i, 0)),
        out_shape=jax.ShapeDtypeStruct((N_NODES, D_OUT), jnp.float32),
    )(P, g, dp, b2)


def kernel(x, edge_index, W, b):
    ei = edge_index.astype(jnp.int32)
    src = ei[0]
    dst = ei[1]
    dp = _deg(dst)
    g = _mm(x, W, dp)
    P = _agg(g, src, dst)
    return _fin(P, g, dp, b.reshape(1, D_OUT))
